# speedup vs baseline: 1.0748x; 1.0748x over previous
import functools

import jax
import jax.numpy as jnp
from jax import lax
from jax.experimental import pallas as pl
from jax.experimental.pallas import tpu as pltpu
from jax.experimental.pallas import tpu_sc as plsc

THETA_MIN = 1.0
THETA_MAX = 5.0
A_MIN = 1.0
A_MAX = 3.0

BATCH = 16384
NC = 2
NS = 16
NW = NC * NS
BPW = BATCH // NW
CHUNK = 128
NCH = BPW // CHUNK
L = 16


def _sigmoid(x):
    return 1.0 / (1.0 + jnp.exp(-x))


_mesh = plsc.VectorSubcoreMesh(core_axis_name="c", subcore_axis_name="s")


@functools.partial(
    pl.kernel,
    mesh=_mesh,
    out_type=jax.ShapeDtypeStruct((BATCH,), jnp.float32),
    scratch_types=[
        pltpu.VMEM((NCH, CHUNK), jnp.int32),
        pltpu.VMEM((NCH, CHUNK), jnp.int32),
        pltpu.VMEM((NCH, CHUNK), jnp.float32),
        pltpu.VMEM((NCH, CHUNK), jnp.float32),
        pltpu.VMEM((NCH, CHUNK), jnp.float32),
        pltpu.VMEM((NCH, CHUNK), jnp.float32),
        pltpu.SemaphoreType.DMA,
    ],
)
def _irt_sc_kernel(theta_hbm, a_hbm, b_hbm, rid_hbm, iid_hbm, out_hbm,
                   rid_v, iid_v, th_v, av_v, bv_v, out_v, sem):
    wid = lax.axis_index("s") * NC + lax.axis_index("c")
    base = wid * BPW

    for j in range(NCH):
        pltpu.sync_copy(rid_hbm.at[pl.ds(base + j * CHUNK, CHUNK)], rid_v.at[j])
        pltpu.sync_copy(iid_hbm.at[pl.ds(base + j * CHUNK, CHUNK)], iid_v.at[j])

    copies = []
    for j in range(NCH):
        copies.append(pltpu.async_copy(theta_hbm.at[rid_v.at[j]], th_v.at[j], sem))
        copies.append(pltpu.async_copy(a_hbm.at[iid_v.at[j]], av_v.at[j], sem))
        copies.append(pltpu.async_copy(b_hbm.at[iid_v.at[j]], bv_v.at[j], sem))
    for c in copies:
        c.wait()

    for j in range(NCH):
        for i in range(CHUNK // L):
            s = pl.ds(i * L, L)
            theta = _sigmoid(th_v[j, s]) * (THETA_MAX - THETA_MIN) + THETA_MIN
            item_a = _sigmoid(av_v[j, s]) * (A_MAX - A_MIN) + A_MIN
            item_b = _sigmoid(bv_v[j, s]) * (THETA_MAX - THETA_MIN) + THETA_MIN
            out_v[j, s] = _sigmoid(item_a * (theta - item_b))

    for j in range(NCH):
        pltpu.sync_copy(out_v.at[j], out_hbm.at[pl.ds(base + j * CHUNK, CHUNK)])


def kernel(respondent_ids, item_ids, a_raw, b_raw, theta_raw):
    rid = respondent_ids.astype(jnp.int32)
    iid = item_ids.astype(jnp.int32)
    return _irt_sc_kernel(
        theta_raw.reshape(-1),
        a_raw.reshape(-1),
        b_raw.reshape(-1),
        rid,
        iid,
    )


# speedup vs baseline: 1.1289x; 1.0503x over previous
import functools

import jax
import jax.numpy as jnp
from jax import lax
from jax.experimental import pallas as pl
from jax.experimental.pallas import tpu as pltpu
from jax.experimental.pallas import tpu_sc as plsc

THETA_MIN = 1.0
THETA_MAX = 5.0
A_MIN = 1.0
A_MAX = 3.0

BATCH = 16384
NC = 2
NS = 16
NW = NC * NS
BPW = BATCH // NW
CHUNK = 128
NCH = BPW // CHUNK
L = 16


def _sigmoid(x):
    return 1.0 / (1.0 + jnp.exp(-x))


_mesh = plsc.VectorSubcoreMesh(core_axis_name="c", subcore_axis_name="s")


@functools.partial(
    pl.kernel,
    mesh=_mesh,
    out_type=jax.ShapeDtypeStruct((NW * NCH, CHUNK), jnp.float32),
    scratch_types=[
        pltpu.VMEM((NCH, CHUNK), jnp.int32),
        pltpu.VMEM((NCH, CHUNK), jnp.int32),
        pltpu.VMEM((NCH, CHUNK), jnp.float32),
        pltpu.VMEM((NCH, CHUNK), jnp.float32),
        pltpu.VMEM((NCH, CHUNK), jnp.float32),
        pltpu.VMEM((NCH, CHUNK), jnp.float32),
        pltpu.SemaphoreType.DMA,
        pltpu.SemaphoreType.DMA,
    ],
)
def _irt_sc_kernel(theta_hbm, a_hbm, b_hbm, rid_hbm, iid_hbm, out_hbm,
                   rid_v, iid_v, th_v, av_v, bv_v, out_v, isem, gsem):
    wid = lax.axis_index("s") * NC + lax.axis_index("c")
    row0 = wid * NCH

    rcp = pltpu.async_copy(rid_hbm.at[pl.ds(row0, NCH)], rid_v, isem)
    icp = pltpu.async_copy(iid_hbm.at[pl.ds(row0, NCH)], iid_v, isem)
    rcp.wait()
    icp.wait()

    copies = []
    for j in range(NCH):
        copies.append(pltpu.async_copy(theta_hbm.at[rid_v.at[j]], th_v.at[j], gsem))
        copies.append(pltpu.async_copy(a_hbm.at[iid_v.at[j]], av_v.at[j], gsem))
        copies.append(pltpu.async_copy(b_hbm.at[iid_v.at[j]], bv_v.at[j], gsem))
    for c in copies:
        c.wait()

    for j in range(NCH):
        for i in range(CHUNK // L):
            s = pl.ds(i * L, L)
            theta = _sigmoid(th_v[j, s]) * (THETA_MAX - THETA_MIN) + THETA_MIN
            item_a = _sigmoid(av_v[j, s]) * (A_MAX - A_MIN) + A_MIN
            item_b = _sigmoid(bv_v[j, s]) * (THETA_MAX - THETA_MIN) + THETA_MIN
            out_v[j, s] = _sigmoid(item_a * (theta - item_b))

    pltpu.sync_copy(out_v, out_hbm.at[pl.ds(row0, NCH)])


def kernel(respondent_ids, item_ids, a_raw, b_raw, theta_raw):
    rid = respondent_ids.astype(jnp.int32).reshape(NW * NCH, CHUNK)
    iid = item_ids.astype(jnp.int32).reshape(NW * NCH, CHUNK)
    out = _irt_sc_kernel(
        theta_raw.reshape(-1),
        a_raw.reshape(-1),
        b_raw.reshape(-1),
        rid,
        iid,
    )
    return out.reshape(-1)
